# baseline (speedup 1.0000x reference)
import jax
import jax.numpy as jnp
from jax.experimental import pallas as pl
from jax.experimental.pallas import tpu as pltpu

TILE = 128
F_BLK = 1024
N_F = 4096 // F_BLK


def _moe_body(tid_ref, eid_ref, gs_ref, ge_ref, first_ref,
              x_ref, wg_ref, wu_ref, wd_ref, out_ref):
    f = pl.program_id(0)
    i = pl.program_id(1)
    xb = x_ref[...]
    g = jnp.dot(xb, wg_ref[0], preferred_element_type=jnp.float32)
    u = jnp.dot(xb, wu_ref[0], preferred_element_type=jnp.float32)
    h = g * jax.nn.sigmoid(g) * u
    rows = tid_ref[i] * TILE + jax.lax.broadcasted_iota(
        jnp.int32, (TILE, 1), 0)
    mask = (rows >= gs_ref[i]) & (rows < ge_ref[i])
    h = jnp.where(mask, h, 0.0)
    y = jnp.dot(h, wd_ref[0], preferred_element_type=jnp.float32)

    base = tid_ref[i] * TILE
    is_first = (f == 0) & (first_ref[i] == 1)

    @pl.when(is_first)
    def _():
        out_ref[pl.ds(base, TILE), :] = y

    @pl.when(jnp.logical_not(is_first))
    def _():
        out_ref[pl.ds(base, TILE), :] += y


def kernel(x, expert_idx, W_gate, W_up, W_down):
    B, S, D = x.shape
    E, _, F = W_gate.shape
    N = B * S
    NT = N // TILE
    G = NT + E - 1
    nf = F // F_BLK

    x_flat = x.reshape(N, D)
    idx = expert_idx.reshape(N).astype(jnp.int32)

    perm = jnp.argsort(idx, stable=True).astype(jnp.int32)
    counts = jnp.bincount(idx, length=E).astype(jnp.int32)
    offsets = jnp.concatenate(
        [jnp.zeros((1,), jnp.int32), jnp.cumsum(counts).astype(jnp.int32)])
    first_tile = offsets[:E] // TILE
    last_tile = jnp.where(counts > 0, (offsets[1:] - 1) // TILE, first_tile)
    ntiles_e = jnp.where(counts > 0, last_tile - first_tile + 1, 0)
    cum = jnp.cumsum(ntiles_e)
    total = cum[-1]

    j = jnp.arange(G, dtype=jnp.int32)
    jc = jnp.minimum(j, total - 1)
    e_of = jnp.searchsorted(cum, jc, side="right").astype(jnp.int32)
    prev_cum = cum[e_of] - ntiles_e[e_of]
    t_of = (first_tile[e_of] + (jc - prev_cum)).astype(jnp.int32)
    gs = jnp.maximum(offsets[e_of], t_of * TILE)
    ge = jnp.minimum(offsets[e_of + 1], (t_of + 1) * TILE)
    valid = j < total
    gs = jnp.where(valid, gs, 0).astype(jnp.int32)
    ge = jnp.where(valid, ge, 0).astype(jnp.int32)
    first = jnp.concatenate(
        [jnp.ones((1,), jnp.bool_), t_of[1:] != t_of[:-1]]) & valid
    first = first.astype(jnp.int32)

    x_sorted = jnp.take(x_flat, perm, axis=0)

    def xmap(f, i, tid_r, eid_r, gs_r, ge_r, first_r):
        return (tid_r[i], 0)

    def wg_map(f, i, tid_r, eid_r, gs_r, ge_r, first_r):
        return (eid_r[i], 0, f)

    def wd_map(f, i, tid_r, eid_r, gs_r, ge_r, first_r):
        return (eid_r[i], f, 0)

    grid_spec = pltpu.PrefetchScalarGridSpec(
        num_scalar_prefetch=5,
        grid=(nf, G),
        in_specs=[
            pl.BlockSpec((TILE, D), xmap),
            pl.BlockSpec((1, D, F_BLK), wg_map),
            pl.BlockSpec((1, D, F_BLK), wg_map),
            pl.BlockSpec((1, F_BLK, D), wd_map),
        ],
        out_specs=pl.BlockSpec(
            (N, D), lambda f, i, *refs: (0, 0)),
    )
    out_sorted = pl.pallas_call(
        _moe_body,
        grid_spec=grid_spec,
        out_shape=jax.ShapeDtypeStruct((N, D), jnp.float32),
        compiler_params=pltpu.CompilerParams(
            dimension_semantics=("arbitrary", "arbitrary"),
        ),
    )(t_of, e_of, gs, ge, first, x_sorted, W_gate, W_up, W_down)

    out_flat = jnp.zeros_like(x_flat).at[perm].set(out_sorted)
    return out_flat.reshape(B, S, D)


# speedup vs baseline: 1.1497x; 1.1497x over previous
import jax
import jax.numpy as jnp
from jax import lax
from jax.experimental import pallas as pl
from jax.experimental.pallas import tpu as pltpu
from jax.experimental.pallas import tpu_sc as plsc

N = 4096
D = 1024
F = 4096
E = 8
TILE = 256
F_BLK = 1024
NT = N // TILE
G = NT + E - 1
NF = F // F_BLK

NW = 32
TPW = N // NW
NCH = N // 16
WCH = TPW // 16

_I16 = (16,)


def _iota16():
    return lax.broadcasted_iota(jnp.int32, _I16, 0)


def _gather16(x, idx):
    return x[idx]


def _cumsum16(x):
    iota = _iota16()
    zeros = jnp.zeros(_I16, jnp.int32)
    for sh in (1, 2, 4, 8):
        shv = jnp.full(_I16, sh, jnp.int32)
        g = _gather16(x, jnp.maximum(iota - shv, zeros))
        x = x + jnp.where(iota >= shv, g, zeros)
    return x


def _splat_lane(x, lane):
    return _gather16(x, jnp.full(_I16, lane, jnp.int32))


def _sc_route_body(idx_hbm, x_hbm, xs_hbm, dest_hbm, meta_hbm,
                   idx_v, dest_v, sidx_v, xbuf_v, meta_v, sem):
    nc = 2
    wid = lax.axis_index("s") * nc + lax.axis_index("c")
    my_first_chunk = wid * WCH

    pltpu.sync_copy(idx_hbm, idx_v)

    zeros = jnp.zeros(_I16, jnp.int32)
    ones = jnp.ones(_I16, jnp.int32)

    def hist_step(c, carry):
        v = idx_v[pl.ds(c * 16, 16)]
        out = []
        for e in range(E):
            m = v == jnp.full(_I16, e, jnp.int32)
            out.append(carry[e] + jnp.where(m, ones, zeros))
        return tuple(out)

    tot_vec = lax.fori_loop(0, NCH, hist_step, (zeros,) * E)
    pre_vec = lax.fori_loop(0, my_first_chunk, hist_step, (zeros,) * E)

    tot_s = [_splat_lane(_cumsum16(tot_vec[e]), 15) for e in range(E)]
    pre_s = [_splat_lane(_cumsum16(pre_vec[e]), 15) for e in range(E)]
    off_s = [zeros]
    for e in range(E):
        off_s.append(off_s[e] + tot_s[e])
    base = [off_s[e] + pre_s[e] for e in range(E)]

    for cc in range(WCH):
        v = idx_v[pl.ds((my_first_chunk + cc) * 16, 16)]
        d = zeros
        for e in range(E):
            m = v == jnp.full(_I16, e, jnp.int32)
            cm = jnp.where(m, ones, zeros)
            incl = _cumsum16(cm)
            d = jnp.where(m, base[e] + incl - ones, d)
            base[e] = base[e] + _splat_lane(incl, 15)
        dest_v[pl.ds(cc * 16, 16)] = d
        sidx_v[cc // 2, pl.ds((cc % 2) * 16, 16)] = d

    pltpu.sync_copy(dest_v, dest_hbm.at[pl.ds(wid * TPW, TPW)])

    handles = []
    for k in range(TPW // 32):
        b = k % 2
        if k >= 2:
            handles[k - 2].wait()
        pltpu.sync_copy(x_hbm.at[pl.ds(wid * TPW + k * 32, 32)],
                        xbuf_v.at[b])
        handles.append(
            pltpu.async_copy(xbuf_v.at[b], xs_hbm.at[sidx_v.at[k]], sem))
    handles[-2].wait()
    handles[-1].wait()

    @pl.when(wid == 0)
    def _():
        eight = jnp.full(_I16, 8, jnp.int32)
        tile_v = jnp.full(_I16, TILE, jnp.int32)
        ft = []
        ntl = []
        for e in range(E):
            has = tot_s[e] > zeros
            ft_e = lax.shift_right_logical(off_s[e], eight)
            lt_e = jnp.where(
                has, lax.shift_right_logical(off_s[e + 1] - ones, eight),
                ft_e)
            ft.append(ft_e)
            ntl.append(jnp.where(has, lt_e - ft_e + ones, zeros))
        cumt = []
        run = zeros
        for e in range(E):
            run = run + ntl[e]
            cumt.append(run)
        total_items = cumt[E - 1]

        def item_fields(jl):
            jcv = jnp.minimum(jl, total_items - ones)
            tv = zeros
            ev = zeros
            gsv = zeros
            gev = zeros
            cprev = zeros
            for e in range(E):
                lo_ok = jcv >= cprev
                hi_ok = jcv < cumt[e]
                ind = jnp.where(lo_ok, jnp.where(hi_ok, ones, zeros), zeros)
                sel = ind > zeros
                t_e = ft[e] + (jcv - cprev)
                gs_e = jnp.maximum(off_s[e], t_e * tile_v)
                ge_e = jnp.minimum(off_s[e + 1], (t_e + ones) * tile_v)
                tv = jnp.where(sel, t_e, tv)
                ev = jnp.where(sel, jnp.full(_I16, e, jnp.int32), ev)
                gsv = jnp.where(sel, gs_e, gsv)
                gev = jnp.where(sel, ge_e, gev)
                cprev = cumt[e]
            return tv, ev, gsv, gev

        for h in range(2):
            jl = _iota16() + jnp.full(_I16, 16 * h, jnp.int32)
            valid = jl < total_items
            tv, ev, gsv, gev = item_fields(jl)
            tprev, _, _, _ = item_fields(jnp.maximum(jl - ones, zeros))
            newt = jnp.where(tv != tprev, ones,
                             jnp.where(jl == zeros, ones, zeros))
            firstv = jnp.where(valid, newt, zeros)
            gsv = jnp.where(valid, gsv, zeros)
            gev = jnp.where(valid, gev, zeros)
            meta_v[0, pl.ds(h * 16, 16)] = tv
            meta_v[1, pl.ds(h * 16, 16)] = ev
            meta_v[2, pl.ds(h * 16, 16)] = gsv
            meta_v[3, pl.ds(h * 16, 16)] = gev
            meta_v[4, pl.ds(h * 16, 16)] = firstv
            meta_v[5, pl.ds(h * 16, 16)] = zeros
            meta_v[6, pl.ds(h * 16, 16)] = zeros
            meta_v[7, pl.ds(h * 16, 16)] = zeros
        pltpu.sync_copy(meta_v, meta_hbm)


def _sc_route(idx, x):
    mesh = plsc.VectorSubcoreMesh(core_axis_name="c", subcore_axis_name="s")
    return pl.kernel(
        _sc_route_body,
        out_type=(
            jax.ShapeDtypeStruct((N, D), jnp.float32),
            jax.ShapeDtypeStruct((N,), jnp.int32),
            jax.ShapeDtypeStruct((8, 32), jnp.int32),
        ),
        mesh=mesh,
        scratch_types=[
            pltpu.VMEM((N,), jnp.int32),
            pltpu.VMEM((TPW,), jnp.int32),
            pltpu.VMEM((TPW // 32, 32), jnp.int32),
            pltpu.VMEM((2, 32, D), jnp.float32),
            pltpu.VMEM((8, 32), jnp.int32),
            pltpu.SemaphoreType.DMA,
        ],
    )(idx, x)


def _sc_combine_body(ys_hbm, dest_hbm, out_hbm, didx_v, rbuf_v, sem):
    nc = 2
    wid = lax.axis_index("s") * nc + lax.axis_index("c")
    pltpu.sync_copy(dest_hbm.at[pl.ds(wid * (TPW // 32), TPW // 32)], didx_v)
    nk = TPW // 32
    handles = []
    for k in range(nk):
        b = k % 2
        if k >= 2:
            handles[k - 2].wait()
            pltpu.sync_copy(rbuf_v.at[b],
                            out_hbm.at[pl.ds(wid * TPW + (k - 2) * 32, 32)])
        handles.append(
            pltpu.async_copy(ys_hbm.at[didx_v.at[k]], rbuf_v.at[b], sem))
    for k in range(nk - 2, nk):
        handles[k].wait()
        pltpu.sync_copy(rbuf_v.at[k % 2],
                        out_hbm.at[pl.ds(wid * TPW + k * 32, 32)])


def _sc_combine(ys, dest):
    mesh = plsc.VectorSubcoreMesh(core_axis_name="c", subcore_axis_name="s")
    dest2 = dest.reshape(N // 32, 32)
    return pl.kernel(
        _sc_combine_body,
        out_type=jax.ShapeDtypeStruct((N, D), jnp.float32),
        mesh=mesh,
        scratch_types=[
            pltpu.VMEM((TPW // 32, 32), jnp.int32),
            pltpu.VMEM((2, 32, D), jnp.float32),
            pltpu.SemaphoreType.DMA,
        ],
    )(ys, dest2)


def _moe_body(tid_ref, eid_ref, gs_ref, ge_ref, first_ref,
              x_ref, wg_ref, wu_ref, wd_ref, out_ref):
    f = pl.program_id(0)
    i = pl.program_id(1)
    xb = x_ref[...]
    g = jnp.dot(xb, wg_ref[0], preferred_element_type=jnp.float32)
    u = jnp.dot(xb, wu_ref[0], preferred_element_type=jnp.float32)
    h = g * jax.nn.sigmoid(g) * u
    rows = tid_ref[i] * TILE + jax.lax.broadcasted_iota(
        jnp.int32, (TILE, 1), 0)
    mask = (rows >= gs_ref[i]) & (rows < ge_ref[i])
    h = jnp.where(mask, h, 0.0)
    y = jnp.dot(h, wd_ref[0], preferred_element_type=jnp.float32)

    base = tid_ref[i] * TILE
    is_first = (f == 0) & (first_ref[i] == 1)

    @pl.when(is_first)
    def _():
        out_ref[pl.ds(base, TILE), :] = y

    @pl.when(jnp.logical_not(is_first))
    def _():
        out_ref[pl.ds(base, TILE), :] += y


def _grouped_swiglu(x_sorted, meta, W_gate, W_up, W_down):
    t_of = meta[0, :G]
    e_of = meta[1, :G]
    gs = meta[2, :G]
    ge = meta[3, :G]
    first = meta[4, :G]

    def xmap(f, i, tid_r, eid_r, gs_r, ge_r, first_r):
        return (tid_r[i], 0)

    def wg_map(f, i, tid_r, eid_r, gs_r, ge_r, first_r):
        return (eid_r[i], 0, f)

    def wd_map(f, i, tid_r, eid_r, gs_r, ge_r, first_r):
        return (eid_r[i], f, 0)

    grid_spec = pltpu.PrefetchScalarGridSpec(
        num_scalar_prefetch=5,
        grid=(NF, G),
        in_specs=[
            pl.BlockSpec((TILE, D), xmap),
            pl.BlockSpec((1, D, F_BLK), wg_map),
            pl.BlockSpec((1, D, F_BLK), wg_map),
            pl.BlockSpec((1, F_BLK, D), wd_map),
        ],
        out_specs=pl.BlockSpec((N, D), lambda f, i, *refs: (0, 0)),
    )
    return pl.pallas_call(
        _moe_body,
        grid_spec=grid_spec,
        out_shape=jax.ShapeDtypeStruct((N, D), jnp.float32),
        compiler_params=pltpu.CompilerParams(
            dimension_semantics=("arbitrary", "arbitrary"),
        ),
    )(t_of, e_of, gs, ge, first, x_sorted, W_gate, W_up, W_down)


def kernel(x, expert_idx, W_gate, W_up, W_down):
    B, S, _ = x.shape
    x_flat = x.reshape(N, D)
    idx = expert_idx.reshape(N).astype(jnp.int32)

    x_sorted, dest, meta = _sc_route(idx, x_flat)
    out_sorted = _grouped_swiglu(x_sorted, meta, W_gate, W_up, W_down)
    out_flat = _sc_combine(out_sorted, dest)
    return out_flat.reshape(B, S, D)
